# baseline (speedup 1.0000x reference)
import functools

import jax
import jax.numpy as jnp
from jax.experimental import pallas as pl
from jax.experimental.pallas import tpu as pltpu

N_NODES = 100000
IN_C = 128
OUT_C = 128
DENSE_OUT = 256
EMB_DIM = 64

ROW_BLOCK = 4000
GRID = N_NODES // ROW_BLOCK


def _gnn_kernel(x_ref, w0_ref, w1_ref, w2_ref, cb_ref, dw_ref, db_ref,
                ew_ref, eb_ref, y_ref, acc_ref, x0_ref):
    i = pl.program_id(0)

    @pl.when(i == 0)
    def _init():
        acc_ref[...] = jnp.zeros_like(acc_ref)
        x0_ref[...] = x_ref[0:1, :]

    acc_ref[...] += jnp.sum(x_ref[...], axis=0, keepdims=True)

    @pl.when(i == GRID - 1)
    def _finish():
        inv_n = 1.0 / N_NODES
        colmean = acc_ref[...] * inv_n
        x0 = x0_ref[...]
        w_mean = w0_ref[...] - w2_ref[...]
        w_corr = 18.0 * w2_ref[...] - 3.0 * w1_ref[...]
        dn = (((1,), (1,)), ((), ()))
        pooled = (
            jax.lax.dot_general(colmean, w_mean, dn,
                                preferred_element_type=jnp.float32)
            + inv_n * jax.lax.dot_general(x0, w_corr, dn,
                                          preferred_element_type=jnp.float32)
            + cb_ref[...]
        )
        h = jax.lax.dot_general(pooled, dw_ref[...], dn,
                                preferred_element_type=jnp.float32) + db_ref[...]
        y = jax.lax.dot_general(h, ew_ref[...], dn,
                                preferred_element_type=jnp.float32) + eb_ref[...]
        y_ref[...] = y


@jax.jit
def kernel(x, cheb_W0, cheb_W1, cheb_W2, cheb_b, dense_W, dense_b, emb_W,
           emb_b):
    cb = cheb_b.reshape(1, OUT_C)
    db = dense_b.reshape(1, DENSE_OUT)
    eb = emb_b.reshape(1, EMB_DIM)

    full = lambda shape: pl.BlockSpec(shape, lambda i: (0,) * len(shape))
    return pl.pallas_call(
        _gnn_kernel,
        grid=(GRID,),
        in_specs=[
            pl.BlockSpec((ROW_BLOCK, IN_C), lambda i: (i, 0)),
            full((OUT_C, IN_C)),
            full((OUT_C, IN_C)),
            full((OUT_C, IN_C)),
            full((1, OUT_C)),
            full((DENSE_OUT, OUT_C)),
            full((1, DENSE_OUT)),
            full((EMB_DIM, DENSE_OUT)),
            full((1, EMB_DIM)),
        ],
        out_specs=pl.BlockSpec((1, EMB_DIM), lambda i: (0, 0)),
        out_shape=jax.ShapeDtypeStruct((1, EMB_DIM), jnp.float32),
        scratch_shapes=[
            pltpu.VMEM((1, IN_C), jnp.float32),
            pltpu.VMEM((1, IN_C), jnp.float32),
        ],
    )(x, cheb_W0, cheb_W1, cheb_W2, cb, dense_W, db, emb_W, eb)


# speedup vs baseline: 1.4234x; 1.4234x over previous
import functools

import jax
import jax.numpy as jnp
from jax.experimental import pallas as pl
from jax.experimental.pallas import tpu as pltpu

N_NODES = 100000
IN_C = 128
OUT_C = 128
DENSE_OUT = 256
EMB_DIM = 64

ROW_BLOCK = 25000
GRID = N_NODES // ROW_BLOCK


def _gnn_kernel(x_ref, w0_ref, w1_ref, w2_ref, cb_ref, dw_ref, db_ref,
                ew_ref, eb_ref, y_ref, acc_ref, x0_ref):
    i = pl.program_id(0)

    @pl.when(i == 0)
    def _init():
        acc_ref[...] = jnp.zeros_like(acc_ref)
        x0_ref[...] = x_ref[0:1, :]

    acc_ref[...] += jnp.sum(x_ref[...], axis=0, keepdims=True)

    @pl.when(i == GRID - 1)
    def _finish():
        inv_n = 1.0 / N_NODES
        colmean = acc_ref[...] * inv_n
        x0 = x0_ref[...]
        w_mean = w0_ref[...] - w2_ref[...]
        w_corr = 18.0 * w2_ref[...] - 3.0 * w1_ref[...]
        dn = (((1,), (1,)), ((), ()))
        pooled = (
            jax.lax.dot_general(colmean, w_mean, dn,
                                preferred_element_type=jnp.float32)
            + inv_n * jax.lax.dot_general(x0, w_corr, dn,
                                          preferred_element_type=jnp.float32)
            + cb_ref[...]
        )
        h = jax.lax.dot_general(pooled, dw_ref[...], dn,
                                preferred_element_type=jnp.float32) + db_ref[...]
        y = jax.lax.dot_general(h, ew_ref[...], dn,
                                preferred_element_type=jnp.float32) + eb_ref[...]
        y_ref[...] = y


@jax.jit
def kernel(x, cheb_W0, cheb_W1, cheb_W2, cheb_b, dense_W, dense_b, emb_W,
           emb_b):
    cb = cheb_b.reshape(1, OUT_C)
    db = dense_b.reshape(1, DENSE_OUT)
    eb = emb_b.reshape(1, EMB_DIM)

    full = lambda shape: pl.BlockSpec(shape, lambda i: (0,) * len(shape))
    return pl.pallas_call(
        _gnn_kernel,
        grid=(GRID,),
        in_specs=[
            pl.BlockSpec((ROW_BLOCK, IN_C), lambda i: (i, 0)),
            full((OUT_C, IN_C)),
            full((OUT_C, IN_C)),
            full((OUT_C, IN_C)),
            full((1, OUT_C)),
            full((DENSE_OUT, OUT_C)),
            full((1, DENSE_OUT)),
            full((EMB_DIM, DENSE_OUT)),
            full((1, EMB_DIM)),
        ],
        out_specs=pl.BlockSpec((1, EMB_DIM), lambda i: (0, 0)),
        out_shape=jax.ShapeDtypeStruct((1, EMB_DIM), jnp.float32),
        scratch_shapes=[
            pltpu.VMEM((1, IN_C), jnp.float32),
            pltpu.VMEM((1, IN_C), jnp.float32),
        ],
    )(x, cheb_W0, cheb_W1, cheb_W2, cb, dense_W, db, emb_W, eb)
